# baseline (speedup 1.0000x reference)
import functools

import jax
import jax.numpy as jnp
from jax import lax
from jax.experimental import pallas as pl
from jax.experimental.pallas import tpu as pltpu
from jax.experimental.pallas import tpu_sc as plsc

N_NODES = 20000
NPAD = 20480
T = 8
E = 320000
ROWS = 2560
EPAD = ROWS * 128
DUMMY = N_NODES
R_PT = ROWS // 32
NODES_PT = NPAD // 16
BN = 2048


def _make_edge_pass(D):
    mesh = plsc.VectorSubcoreMesh(core_axis_name="c", subcore_axis_name="s")

    @functools.partial(
        pl.kernel,
        mesh=mesh,
        out_type=jax.ShapeDtypeStruct((2, NPAD, D), jnp.float32),
        scratch_types=[
            pltpu.VMEM((R_PT, 128), jnp.int32),
            pltpu.VMEM((R_PT, 128), jnp.int32),
            pltpu.VMEM((128, D), jnp.float32),
            pltpu.VMEM_SHARED((NPAD, D), jnp.float32),
            pltpu.SemaphoreType.DMA,
        ],
        compiler_params=pltpu.CompilerParams(use_tc_tiling_on_sc=False),
    )
    def edge_pass(src_hbm, dst_hbm, tab_hbm, zero_hbm, out_hbm,
                  srcv, dstv, rows, acc, sem):
        c = lax.axis_index("c")
        s = lax.axis_index("s")
        nbase = s * NODES_PT
        pltpu.sync_copy(zero_hbm.at[pl.ds(nbase, NODES_PT)],
                        acc.at[pl.ds(nbase, NODES_PT)])
        rbase = (c * 16 + s) * R_PT
        pltpu.sync_copy(src_hbm.at[pl.ds(rbase, R_PT)], srcv)
        pltpu.sync_copy(dst_hbm.at[pl.ds(rbase, R_PT)], dstv)
        plsc.subcore_barrier()

        def body(j, carry):
            pltpu.async_copy(tab_hbm.at[srcv.at[j]], rows, sem).wait()
            pltpu.sync_copy(rows, acc.at[dstv.at[j]], add=True)
            return carry

        lax.fori_loop(0, R_PT, body, 0)
        plsc.subcore_barrier()
        pltpu.sync_copy(acc.at[pl.ds(nbase, NODES_PT)],
                        out_hbm.at[c, pl.ds(nbase, NODES_PT)])

    return edge_pass


def _make_deg_pass():
    mesh = plsc.VectorSubcoreMesh(core_axis_name="c", subcore_axis_name="s")

    @functools.partial(
        pl.kernel,
        mesh=mesh,
        out_type=jax.ShapeDtypeStruct((2, NPAD, 8), jnp.float32),
        scratch_types=[
            pltpu.VMEM((R_PT, 128), jnp.int32),
            pltpu.VMEM((128, 8), jnp.float32),
            pltpu.VMEM_SHARED((NPAD, 8), jnp.float32),
            pltpu.SemaphoreType.DMA,
        ],
        compiler_params=pltpu.CompilerParams(use_tc_tiling_on_sc=False),
    )
    def deg_pass(dst_hbm, ones_hbm, zero_hbm, out_hbm, dstv, onesv, acc, sem):
        c = lax.axis_index("c")
        s = lax.axis_index("s")
        nbase = s * NODES_PT
        pltpu.sync_copy(zero_hbm.at[pl.ds(nbase, NODES_PT)],
                        acc.at[pl.ds(nbase, NODES_PT)])
        rbase = (c * 16 + s) * R_PT
        pltpu.sync_copy(dst_hbm.at[pl.ds(rbase, R_PT)], dstv)
        pltpu.sync_copy(ones_hbm, onesv)
        plsc.subcore_barrier()

        def body(j, carry):
            pltpu.sync_copy(onesv, acc.at[dstv.at[j]], add=True)
            return carry

        lax.fori_loop(0, R_PT, body, 0)
        plsc.subcore_barrier()
        pltpu.sync_copy(acc.at[pl.ds(nbase, NODES_PT)],
                        out_hbm.at[c, pl.ds(nbase, NODES_PT)])

    return deg_pass


def _prep1(d0, d1, xp):
    def body(d0_ref, d1_ref, x_ref, dinv_ref, s1_ref):
        deg = 1.0 + d0_ref[:, 0:1] + d1_ref[:, 0:1]
        dinv = lax.rsqrt(deg)
        dinv_ref[...] = dinv
        s1_ref[...] = dinv * x_ref[...]

    return pl.pallas_call(
        body,
        grid=(NPAD // BN,),
        in_specs=[pl.BlockSpec((BN, 8), lambda i: (i, 0))] * 3,
        out_specs=[pl.BlockSpec((BN, 1), lambda i: (i, 0)),
                   pl.BlockSpec((BN, 8), lambda i: (i, 0))],
        out_shape=[jax.ShapeDtypeStruct((NPAD, 1), jnp.float32),
                   jax.ShapeDtypeStruct((NPAD, 8), jnp.float32)],
    )(d0, d1, xp)


def _prep2(q0, q1, s1, dinv):
    def body(q0_ref, q1_ref, s1_ref, dinv_ref, s2_ref):
        dinv = dinv_ref[...]
        u = dinv * (q0_ref[...] + q1_ref[...] + s1_ref[...])
        ap = jnp.maximum(u, 0.0)
        am = u - ap
        s2_ref[:, :8] = dinv * ap
        s2_ref[:, 8:] = dinv * am

    return pl.pallas_call(
        body,
        grid=(NPAD // BN,),
        in_specs=[pl.BlockSpec((BN, 8), lambda i: (i, 0))] * 3
                 + [pl.BlockSpec((BN, 1), lambda i: (i, 0))],
        out_specs=pl.BlockSpec((BN, 16), lambda i: (i, 0)),
        out_shape=jax.ShapeDtypeStruct((NPAD, 16), jnp.float32),
    )(q0, q1, s1, dinv)


def _lstm(p0, p1, s2, dinv, W1, W2, W_ihT, W_hhT, bsum, b2r, fc_W, fc_b):
    def body(p0_ref, p1_ref, s2_ref, dinv_ref, w1_ref, w2_ref, wih_ref,
             whh_ref, bsum_ref, b2_ref, fcw_ref, fcb_ref, out_ref):
        f32 = jnp.float32
        w1 = w1_ref[...]
        cp = jnp.dot(jnp.maximum(w1, 0.0), w2_ref[...],
                     preferred_element_type=f32,
                     precision=lax.Precision.HIGHEST)
        cm = jnp.dot(jnp.minimum(w1, 0.0), w2_ref[...],
                     preferred_element_type=f32,
                     precision=lax.Precision.HIGHEST)
        wih = wih_ref[...]
        gp = jnp.dot(cp, wih, preferred_element_type=f32,
                     precision=lax.Precision.HIGHEST)
        gm = jnp.dot(cm, wih, preferred_element_type=f32,
                     precision=lax.Precision.HIGHEST)
        g0 = jnp.dot(b2_ref[...], wih, preferred_element_type=f32,
                     precision=lax.Precision.HIGHEST) + bsum_ref[...]
        whh = whh_ref[...]

        V = dinv_ref[...] * (p0_ref[...] + p1_ref[...] + s2_ref[...])
        h = jnp.zeros((BN, 64), f32)
        c = jnp.zeros((BN, 64), f32)
        for t in range(T):
            gates = (V[:, t:t + 1] * gp + V[:, 8 + t:9 + t] * gm + g0
                     + jnp.dot(h, whh, preferred_element_type=f32,
                     precision=lax.Precision.HIGHEST))
            i = jax.nn.sigmoid(gates[:, :64])
            f = jax.nn.sigmoid(gates[:, 64:128])
            g = jnp.tanh(gates[:, 128:192])
            o = jax.nn.sigmoid(gates[:, 192:256])
            c = f * c + i * g
            h = o * jnp.tanh(c)
        out_ref[...] = jnp.dot(h, fcw_ref[...],
                               preferred_element_type=f32,
                     precision=lax.Precision.HIGHEST) + fcb_ref[...]

    node = lambda w: pl.BlockSpec((BN, w), lambda i: (i, 0))
    full = lambda a: pl.BlockSpec(a.shape, lambda i: (0, 0))
    return pl.pallas_call(
        body,
        grid=(NPAD // BN,),
        in_specs=[node(16), node(16), node(16), node(1),
                  full(W1), full(W2), full(W_ihT), full(W_hhT),
                  full(bsum), full(b2r), full(fc_W), full(fc_b)],
        out_specs=node(1),
        out_shape=jax.ShapeDtypeStruct((NPAD, 1), jnp.float32),
    )(p0, p1, s2, dinv, W1, W2, W_ihT, W_hhT, bsum, b2r, fc_W, fc_b)


def kernel(x, edge_index, W1, b1, W2, b2, W_ih, W_hh, b_ih, b_hh, fc_W, fc_b):
    B, T_, NN = x.shape
    X = jnp.transpose(x, (0, 2, 1)).reshape(B * NN, T_)
    Xp = jnp.pad(X, ((0, NPAD - N_NODES), (0, 0)))

    src = jnp.pad(edge_index[0], (0, EPAD - E), constant_values=DUMMY)
    dst = jnp.pad(edge_index[1], (0, EPAD - E), constant_values=DUMMY)
    src2d = src.reshape(ROWS, 128)
    dst2d = dst.reshape(ROWS, 128)

    ones1 = jnp.ones((128, 8), jnp.float32)
    zeros1 = jnp.zeros((NPAD, 8), jnp.float32)
    zeros8 = jnp.zeros((NPAD, 8), jnp.float32)
    zeros16 = jnp.zeros((NPAD, 16), jnp.float32)

    dpart = _make_deg_pass()(dst2d, ones1, zeros1)
    dinv, S1 = _prep1(dpart[0], dpart[1], Xp)
    qpart = _make_edge_pass(8)(src2d, dst2d, S1, zeros8)
    S2 = _prep2(qpart[0], qpart[1], S1, dinv)
    ppart = _make_edge_pass(16)(src2d, dst2d, S2, zeros16)

    bsum = (b_ih + b_hh).reshape(1, 256)
    out = _lstm(ppart[0], ppart[1], S2, dinv, W1, W2,
                W_ih.T, W_hh.T, bsum, b2.reshape(1, 64),
                fc_W, fc_b.reshape(1, 1))
    return out[:N_NODES, 0]


# speedup vs baseline: 1.0538x; 1.0538x over previous
import functools

import jax
import jax.numpy as jnp
from jax import lax
from jax.experimental import pallas as pl
from jax.experimental.pallas import tpu as pltpu
from jax.experimental.pallas import tpu_sc as plsc

N_NODES = 20000
NPAD = 20480
T = 8
E = 320000
ROWS = 2560
EPAD = ROWS * 128
DUMMY = N_NODES
R_PT = ROWS // 32
NODES_PT = NPAD // 16
BN = 2048


def _make_edge_pass(D):
    mesh = plsc.VectorSubcoreMesh(core_axis_name="c", subcore_axis_name="s")

    @functools.partial(
        pl.kernel,
        mesh=mesh,
        out_type=jax.ShapeDtypeStruct((2, NPAD, D), jnp.float32),
        scratch_types=[
            pltpu.VMEM((R_PT, 128), jnp.int32),
            pltpu.VMEM((R_PT, 128), jnp.int32),
            pltpu.VMEM((128, D), jnp.float32),
            pltpu.VMEM((128, D), jnp.float32),
            pltpu.VMEM_SHARED((NPAD, D), jnp.float32),
            pltpu.SemaphoreType.DMA,
            pltpu.SemaphoreType.DMA,
        ],
        compiler_params=pltpu.CompilerParams(use_tc_tiling_on_sc=False),
    )
    def edge_pass(src_hbm, dst_hbm, tab_hbm, zero_hbm, out_hbm,
                  srcv, dstv, rows_a, rows_b, acc, sem_a, sem_b):
        c = lax.axis_index("c")
        s = lax.axis_index("s")
        nbase = s * NODES_PT
        pltpu.sync_copy(zero_hbm.at[pl.ds(nbase, NODES_PT)],
                        acc.at[pl.ds(nbase, NODES_PT)])
        rbase = (c * 16 + s) * R_PT
        pltpu.sync_copy(src_hbm.at[pl.ds(rbase, R_PT)], srcv)
        pltpu.sync_copy(dst_hbm.at[pl.ds(rbase, R_PT)], dstv)
        plsc.subcore_barrier()

        def body(j, carry):
            ca = pltpu.async_copy(tab_hbm.at[srcv.at[2 * j]], rows_a, sem_a)
            cb = pltpu.async_copy(tab_hbm.at[srcv.at[2 * j + 1]], rows_b, sem_b)
            ca.wait()
            pltpu.sync_copy(rows_a, acc.at[dstv.at[2 * j]], add=True)
            cb.wait()
            pltpu.sync_copy(rows_b, acc.at[dstv.at[2 * j + 1]], add=True)
            return carry

        lax.fori_loop(0, R_PT // 2, body, 0)
        plsc.subcore_barrier()
        pltpu.sync_copy(acc.at[pl.ds(nbase, NODES_PT)],
                        out_hbm.at[c, pl.ds(nbase, NODES_PT)])

    return edge_pass


def _make_deg_pass():
    mesh = plsc.VectorSubcoreMesh(core_axis_name="c", subcore_axis_name="s")

    @functools.partial(
        pl.kernel,
        mesh=mesh,
        out_type=jax.ShapeDtypeStruct((2, NPAD, 8), jnp.float32),
        scratch_types=[
            pltpu.VMEM((R_PT, 128), jnp.int32),
            pltpu.VMEM((128, 8), jnp.float32),
            pltpu.VMEM_SHARED((NPAD, 8), jnp.float32),
            pltpu.SemaphoreType.DMA,
        ],
        compiler_params=pltpu.CompilerParams(use_tc_tiling_on_sc=False),
    )
    def deg_pass(dst_hbm, ones_hbm, zero_hbm, out_hbm, dstv, onesv, acc, sem):
        c = lax.axis_index("c")
        s = lax.axis_index("s")
        nbase = s * NODES_PT
        pltpu.sync_copy(zero_hbm.at[pl.ds(nbase, NODES_PT)],
                        acc.at[pl.ds(nbase, NODES_PT)])
        rbase = (c * 16 + s) * R_PT
        pltpu.sync_copy(dst_hbm.at[pl.ds(rbase, R_PT)], dstv)
        pltpu.sync_copy(ones_hbm, onesv)
        plsc.subcore_barrier()

        def body(j, carry):
            pltpu.sync_copy(onesv, acc.at[dstv.at[j]], add=True)
            return carry

        lax.fori_loop(0, R_PT, body, 0)
        plsc.subcore_barrier()
        pltpu.sync_copy(acc.at[pl.ds(nbase, NODES_PT)],
                        out_hbm.at[c, pl.ds(nbase, NODES_PT)])

    return deg_pass


def _prep1(d0, d1, xp):
    def body(d0_ref, d1_ref, x_ref, dinv_ref, s1_ref):
        deg = 1.0 + d0_ref[:, 0:1] + d1_ref[:, 0:1]
        dinv = lax.rsqrt(deg)
        dinv_ref[...] = dinv
        s1_ref[...] = dinv * x_ref[...]

    return pl.pallas_call(
        body,
        grid=(NPAD // BN,),
        in_specs=[pl.BlockSpec((BN, 8), lambda i: (i, 0))] * 3,
        out_specs=[pl.BlockSpec((BN, 1), lambda i: (i, 0)),
                   pl.BlockSpec((BN, 8), lambda i: (i, 0))],
        out_shape=[jax.ShapeDtypeStruct((NPAD, 1), jnp.float32),
                   jax.ShapeDtypeStruct((NPAD, 8), jnp.float32)],
    )(d0, d1, xp)


def _prep2(q0, q1, s1, dinv):
    def body(q0_ref, q1_ref, s1_ref, dinv_ref, s2_ref):
        dinv = dinv_ref[...]
        u = dinv * (q0_ref[...] + q1_ref[...] + s1_ref[...])
        ap = jnp.maximum(u, 0.0)
        am = u - ap
        s2_ref[:, :8] = dinv * ap
        s2_ref[:, 8:] = dinv * am

    return pl.pallas_call(
        body,
        grid=(NPAD // BN,),
        in_specs=[pl.BlockSpec((BN, 8), lambda i: (i, 0))] * 3
                 + [pl.BlockSpec((BN, 1), lambda i: (i, 0))],
        out_specs=pl.BlockSpec((BN, 16), lambda i: (i, 0)),
        out_shape=jax.ShapeDtypeStruct((NPAD, 16), jnp.float32),
    )(q0, q1, s1, dinv)


def _lstm(p0, p1, s2, dinv, W1, W2, W_ihT, W_hhT, bsum, b2r, fc_W, fc_b):
    def body(p0_ref, p1_ref, s2_ref, dinv_ref, w1_ref, w2_ref, wih_ref,
             whh_ref, bsum_ref, b2_ref, fcw_ref, fcb_ref, out_ref):
        f32 = jnp.float32
        w1 = w1_ref[...]
        cp = jnp.dot(jnp.maximum(w1, 0.0), w2_ref[...],
                     preferred_element_type=f32,
                     precision=lax.Precision.HIGHEST)
        cm = jnp.dot(jnp.minimum(w1, 0.0), w2_ref[...],
                     preferred_element_type=f32,
                     precision=lax.Precision.HIGHEST)
        wih = wih_ref[...]
        gp = jnp.dot(cp, wih, preferred_element_type=f32,
                     precision=lax.Precision.HIGHEST)
        gm = jnp.dot(cm, wih, preferred_element_type=f32,
                     precision=lax.Precision.HIGHEST)
        g0 = jnp.dot(b2_ref[...], wih, preferred_element_type=f32,
                     precision=lax.Precision.HIGHEST) + bsum_ref[...]
        whh = whh_ref[...]

        V = dinv_ref[...] * (p0_ref[...] + p1_ref[...] + s2_ref[...])
        h = jnp.zeros((BN, 64), f32)
        c = jnp.zeros((BN, 64), f32)
        for t in range(T):
            gates = (V[:, t:t + 1] * gp + V[:, 8 + t:9 + t] * gm + g0
                     + jnp.dot(h, whh, preferred_element_type=f32,
                     precision=lax.Precision.HIGHEST))
            i = jax.nn.sigmoid(gates[:, :64])
            f = jax.nn.sigmoid(gates[:, 64:128])
            g = jnp.tanh(gates[:, 128:192])
            o = jax.nn.sigmoid(gates[:, 192:256])
            c = f * c + i * g
            h = o * jnp.tanh(c)
        out_ref[...] = jnp.dot(h, fcw_ref[...],
                               preferred_element_type=f32,
                     precision=lax.Precision.HIGHEST) + fcb_ref[...]

    node = lambda w: pl.BlockSpec((BN, w), lambda i: (i, 0))
    full = lambda a: pl.BlockSpec(a.shape, lambda i: (0, 0))
    return pl.pallas_call(
        body,
        grid=(NPAD // BN,),
        in_specs=[node(16), node(16), node(16), node(1),
                  full(W1), full(W2), full(W_ihT), full(W_hhT),
                  full(bsum), full(b2r), full(fc_W), full(fc_b)],
        out_specs=node(1),
        out_shape=jax.ShapeDtypeStruct((NPAD, 1), jnp.float32),
    )(p0, p1, s2, dinv, W1, W2, W_ihT, W_hhT, bsum, b2r, fc_W, fc_b)


def kernel(x, edge_index, W1, b1, W2, b2, W_ih, W_hh, b_ih, b_hh, fc_W, fc_b):
    B, T_, NN = x.shape
    X = jnp.transpose(x, (0, 2, 1)).reshape(B * NN, T_)
    Xp = jnp.pad(X, ((0, NPAD - N_NODES), (0, 0)))

    src = jnp.pad(edge_index[0], (0, EPAD - E), constant_values=DUMMY)
    dst = jnp.pad(edge_index[1], (0, EPAD - E), constant_values=DUMMY)
    src2d = src.reshape(ROWS, 128)
    dst2d = dst.reshape(ROWS, 128)

    ones1 = jnp.ones((128, 8), jnp.float32)
    zeros1 = jnp.zeros((NPAD, 8), jnp.float32)
    zeros8 = jnp.zeros((NPAD, 8), jnp.float32)
    zeros16 = jnp.zeros((NPAD, 16), jnp.float32)

    dpart = _make_deg_pass()(dst2d, ones1, zeros1)
    dinv, S1 = _prep1(dpart[0], dpart[1], Xp)
    qpart = _make_edge_pass(8)(src2d, dst2d, S1, zeros8)
    S2 = _prep2(qpart[0], qpart[1], S1, dinv)
    ppart = _make_edge_pass(16)(src2d, dst2d, S2, zeros16)

    bsum = (b_ih + b_hh).reshape(1, 256)
    out = _lstm(ppart[0], ppart[1], S2, dinv, W1, W2,
                W_ih.T, W_hh.T, bsum, b2.reshape(1, 64),
                fc_W, fc_b.reshape(1, 1))
    return out[:N_NODES, 0]
